# baseline (speedup 1.0000x reference)
import functools

import jax
import jax.numpy as jnp
from jax import lax
from jax.experimental import pallas as pl
from jax.experimental.pallas import tpu as pltpu
from jax.experimental.pallas import tpu_sc as plsc

_PELVIS = (0.5, 0.25, 0.25)
_NECK = (0.4, 0.3, 0.3)
_SHOULDER = (0.3, 0.2, 0.5)
_HIP = (0.6, 0.2, 0.2)

_L = 16
_NW = 32


def _sc_body(raw_hbm, j14_hbm, op_hbm, out_hbm, op_v, raw_v, j_v):
    n = op_v.shape[0] // 75
    wid = lax.axis_index("s") * 2 + lax.axis_index("c")
    pltpu.sync_copy(op_hbm.at[pl.ds(wid * (n * 75), n * 75)], op_v)
    pltpu.sync_copy(raw_hbm.at[pl.ds(wid * (n * 72), n * 72)], raw_v)
    pltpu.sync_copy(j14_hbm.at[pl.ds(wid * (n * 42), n * 42)], j_v)

    lane = lax.iota(jnp.int32, _L)

    def body(g, carry):
        rows = g * _L + lane
        r72 = rows * 72
        r42 = rows * 42
        r75 = rows * 75

        for c in range(3):
            a0 = plsc.load_gather(raw_v, [r72 + (0 + c)])
            a1 = plsc.load_gather(raw_v, [r72 + (3 + c)])
            a2 = plsc.load_gather(raw_v, [r72 + (6 + c)])
            b0 = plsc.load_gather(raw_v, [r72 + (36 + c)])
            b1 = plsc.load_gather(raw_v, [r72 + (39 + c)])
            b2 = plsc.load_gather(raw_v, [r72 + (42 + c)])
            c0 = plsc.load_gather(raw_v, [r72 + (48 + c)])
            c1 = plsc.load_gather(raw_v, [r72 + (51 + c)])
            j0 = plsc.load_gather(j_v, [r42 + (3 + c)])
            j1 = plsc.load_gather(j_v, [r42 + (12 + c)])

            pelvis = _PELVIS[0] * a0 + _PELVIS[1] * a1 + _PELVIS[2] * a2
            neck = _NECK[0] * b0 + _NECK[1] * b1 + _NECK[2] * b2
            lsh = _SHOULDER[0] * c0 + _SHOULDER[1] * b0 + _SHOULDER[2] * b1
            rsh = _SHOULDER[0] * c1 + _SHOULDER[1] * b0 + _SHOULDER[2] * b2
            lhip = _HIP[0] * a1 + _HIP[1] * a0 + _HIP[2] * j0
            rhip = _HIP[0] * a2 + _HIP[1] * a0 + _HIP[2] * j1

            plsc.store_scatter(op_v, [r75 + (3 + c)], neck)
            plsc.store_scatter(op_v, [r75 + (6 + c)], rsh)
            plsc.store_scatter(op_v, [r75 + (15 + c)], lsh)
            plsc.store_scatter(op_v, [r75 + (24 + c)], pelvis)
            plsc.store_scatter(op_v, [r75 + (27 + c)], rhip)
            plsc.store_scatter(op_v, [r75 + (36 + c)], lhip)
        return carry

    lax.fori_loop(0, n // _L, body, 0)
    pltpu.sync_copy(op_v, out_hbm.at[pl.ds(wid * (n * 75), n * 75)])


def kernel(raw, j14, openpose):
    B = raw.shape[0]
    n = B // _NW

    mesh = plsc.VectorSubcoreMesh(core_axis_name="c", subcore_axis_name="s")
    f = functools.partial(
        pl.kernel,
        mesh=mesh,
        compiler_params=pltpu.CompilerParams(needs_layout_passes=False),
        out_type=jax.ShapeDtypeStruct((B * 75,), jnp.float32),
        scratch_types=[
            pltpu.VMEM((n * 75,), jnp.float32),
            pltpu.VMEM((n * 72,), jnp.float32),
            pltpu.VMEM((n * 42,), jnp.float32),
        ],
    )(_sc_body)
    out = f(raw.reshape(B * 72), j14.reshape(B * 42), openpose.reshape(B * 75))
    return out.reshape(B, 25, 3)


# speedup vs baseline: 26.2597x; 26.2597x over previous
import functools

import jax
import jax.numpy as jnp
from jax import lax
from jax.experimental import pallas as pl
from jax.experimental.pallas import tpu as pltpu
from jax.experimental.pallas import tpu_sc as plsc

_PELVIS = (0.5, 0.25, 0.25)
_NECK = (0.4, 0.3, 0.3)
_SHOULDER = (0.3, 0.2, 0.5)
_HIP = (0.6, 0.2, 0.2)

_L = 16
_NW = 32


def _plane_specs():
    specs = []
    for c in range(3):
        r = lambda jj: c * 24 + jj
        jr = lambda kk: c * 14 + kk
        o = lambda kk: c * 25 + kk
        specs += [
            (o(8), [(_PELVIS[0], "r", r(0)), (_PELVIS[1], "r", r(1)), (_PELVIS[2], "r", r(2))]),
            (o(1), [(_NECK[0], "r", r(12)), (_NECK[1], "r", r(13)), (_NECK[2], "r", r(14))]),
            (o(5), [(_SHOULDER[0], "r", r(16)), (_SHOULDER[1], "r", r(12)), (_SHOULDER[2], "r", r(13))]),
            (o(2), [(_SHOULDER[0], "r", r(17)), (_SHOULDER[1], "r", r(12)), (_SHOULDER[2], "r", r(14))]),
            (o(12), [(_HIP[0], "r", r(1)), (_HIP[1], "r", r(0)), (_HIP[2], "j", jr(1))]),
            (o(9), [(_HIP[0], "r", r(2)), (_HIP[1], "r", r(0)), (_HIP[2], "j", jr(4))]),
        ]
    return specs


def _sc_body(raw_hbm, j14_hbm, op_hbm, out_hbm, op_v, raw_v, j_v):
    m = op_v.shape[1]
    wid = lax.axis_index("s") * 2 + lax.axis_index("c")
    wb = wid * m
    pltpu.sync_copy(op_hbm.at[:, pl.ds(wb, m)], op_v)
    pltpu.sync_copy(raw_hbm.at[:, pl.ds(wb, m)], raw_v)
    pltpu.sync_copy(j14_hbm.at[:, pl.ds(wb, m)], j_v)

    specs = _plane_specs()

    def body(g, carry):
        k = g * _L
        for out_r, terms in specs:
            acc = None
            for w, arr, rr in terms:
                src = raw_v if arr == "r" else j_v
                v = w * src[rr, pl.ds(k, _L)]
                acc = v if acc is None else acc + v
            op_v[out_r, pl.ds(k, _L)] = acc
        return carry

    lax.fori_loop(0, m // _L, body, 0)
    pltpu.sync_copy(op_v, out_hbm.at[:, pl.ds(wb, m)])


def kernel(raw, j14, openpose):
    B = raw.shape[0]
    m = B // _NW

    mesh = plsc.VectorSubcoreMesh(core_axis_name="c", subcore_axis_name="s")
    f = functools.partial(
        pl.kernel,
        mesh=mesh,
        compiler_params=pltpu.CompilerParams(needs_layout_passes=False),
        out_type=jax.ShapeDtypeStruct((75, B), jnp.float32),
        scratch_types=[
            pltpu.VMEM((75, m), jnp.float32),
            pltpu.VMEM((72, m), jnp.float32),
            pltpu.VMEM((42, m), jnp.float32),
        ],
    )(_sc_body)
    out = f(
        raw.transpose(2, 1, 0).reshape(72, B),
        j14.transpose(2, 1, 0).reshape(42, B),
        openpose.transpose(2, 1, 0).reshape(75, B),
    )
    return out.reshape(3, 25, B).transpose(2, 1, 0)


# speedup vs baseline: 39.9109x; 1.5199x over previous
import functools

import jax
import jax.numpy as jnp
from jax import lax
from jax.experimental import pallas as pl
from jax.experimental.pallas import tpu as pltpu
from jax.experimental.pallas import tpu_sc as plsc

_PELVIS = (0.5, 0.25, 0.25)
_NECK = (0.4, 0.3, 0.3)
_SHOULDER = (0.3, 0.2, 0.5)
_HIP = (0.6, 0.2, 0.2)

_L = 16
_NW = 32

_RAW_POS = {j: j for j in range(18)}
_J14_ROWS = 8


def _plane_specs():
    specs = []
    for c in range(3):
        r = lambda jj: _RAW_POS[jj]
        specs += [
            (c, 8, [(_PELVIS[0], "r", r(0)), (_PELVIS[1], "r", r(1)), (_PELVIS[2], "r", r(2))]),
            (c, 1, [(_NECK[0], "r", r(12)), (_NECK[1], "r", r(13)), (_NECK[2], "r", r(14))]),
            (c, 5, [(_SHOULDER[0], "r", r(16)), (_SHOULDER[1], "r", r(12)), (_SHOULDER[2], "r", r(13))]),
            (c, 2, [(_SHOULDER[0], "r", r(17)), (_SHOULDER[1], "r", r(12)), (_SHOULDER[2], "r", r(14))]),
            (c, 12, [(_HIP[0], "r", r(1)), (_HIP[1], "r", r(0)), (_HIP[2], "j", 1)]),
            (c, 9, [(_HIP[0], "r", r(2)), (_HIP[1], "r", r(0)), (_HIP[2], "j", 4)]),
        ]
    return specs


def _sc_body(raw_hbm, j14_hbm, op_hbm, op24_hbm, out_hbm, op_v, raw_v, j_v, op24_v, sem_rj, sem_op):
    m = op_v.shape[2]
    wid = lax.axis_index("s") * 2 + lax.axis_index("c")
    wb = wid * m

    rj_waits = []
    op_waits = []
    rj_waits.append(pltpu.async_copy(
        raw_hbm.at[:, :, pl.ds(wb, m)], raw_v, sem_rj))
    for c in range(3):
        rj_waits.append(pltpu.async_copy(
            j14_hbm.at[c, pl.ds(0, _J14_ROWS), pl.ds(wb, m)],
            j_v.at[c], sem_rj))
        op_waits.append(pltpu.async_copy(
            op_hbm.at[c, pl.ds(0, 24), pl.ds(wb, m)],
            op_v.at[c, pl.ds(0, 24)], sem_op))
        op_waits.append(pltpu.async_copy(
            op24_hbm.at[c, :, pl.ds(wb, m)],
            op24_v.at[c], sem_op))
    for h in rj_waits:
        h.wait()

    specs = _plane_specs()

    def body(g, carry):
        k = g * _L
        for c in range(3):
            op_v[c, 24, pl.ds(k, _L)] = op24_v[c, 0, pl.ds(k, _L)]
        for c, out_j, terms in specs:
            acc = None
            for w, arr, jj in terms:
                src = raw_v if arr == "r" else j_v
                v = w * src[c, jj, pl.ds(k, _L)]
                acc = v if acc is None else acc + v
            op_v[c, out_j, pl.ds(k, _L)] = acc
        return carry

    for h in op_waits:
        h.wait()
    lax.fori_loop(0, m // _L, body, 0)
    pltpu.sync_copy(op_v, out_hbm.at[:, :, pl.ds(wb, m)])


def kernel(raw, j14, openpose):
    B = raw.shape[0]
    m = B // _NW

    mesh = plsc.VectorSubcoreMesh(core_axis_name="c", subcore_axis_name="s")
    f = functools.partial(
        pl.kernel,
        mesh=mesh,
        compiler_params=pltpu.CompilerParams(needs_layout_passes=False),
        out_type=jax.ShapeDtypeStruct((3, 32, B), jnp.float32),
        scratch_types=[
            pltpu.VMEM((3, 32, m), jnp.float32),
            pltpu.VMEM((3, 24, m), jnp.float32),
            pltpu.VMEM((3, _J14_ROWS, m), jnp.float32),
            pltpu.VMEM((3, 1, m), jnp.float32),
            pltpu.SemaphoreType.DMA,
            pltpu.SemaphoreType.DMA,
        ],
    )(_sc_body)
    opT = openpose.transpose(2, 1, 0)
    out = f(
        raw.transpose(2, 1, 0),
        j14.transpose(2, 1, 0),
        opT,
        opT[:, 24:25, :],
    )
    return out.transpose(2, 1, 0)[:, :25, :]
